# Initial kernel scaffold; baseline (speedup 1.0000x reference)
import jax
import jax.numpy as jnp
from jax.experimental import pallas as pl


def kernel(outputs, index):
    raise NotImplementedError("write your pallas kernel here")


# speedup vs baseline: 1.8778x; 1.8778x over previous
import functools

import jax
import jax.numpy as jnp
from jax import lax
from jax.experimental import pallas as pl
from jax.experimental.pallas import tpu as pltpu
from jax.experimental.pallas import tpu_sc as plsc

NC = 2
NS = 16
L = 16
NW = NC * NS

H = 9
NH = 8
D = 128
RW = H * D


def _phase1(x_flat, i_flat, B):
    rows_per_w = B // NW
    ngroups = rows_per_w // L
    gw = L * RW

    mesh = plsc.VectorSubcoreMesh(core_axis_name="c", subcore_axis_name="s")

    @functools.partial(
        pl.kernel,
        out_type=(
            jax.ShapeDtypeStruct((B * NH,), jnp.float32),
            jax.ShapeDtypeStruct((B,), jnp.int32),
            jax.ShapeDtypeStruct((NW * L,), jnp.float32),
            jax.ShapeDtypeStruct((NW * L,), jnp.float32),
        ),
        mesh=mesh,
        scratch_types=[
            pltpu.VMEM((2 * gw,), jnp.float32),
            pltpu.VMEM((rows_per_w * 2,), jnp.int32),
            pltpu.VMEM((rows_per_w * NH,), jnp.float32),
            pltpu.VMEM((rows_per_w,), jnp.int32),
            pltpu.VMEM((L,), jnp.float32),
            pltpu.SemaphoreType.DMA,
        ],
        compiler_params=pltpu.CompilerParams(needs_layout_passes=False),
    )
    def body(x_hbm, i_hbm, diff_hbm, pred_hbm, lpart_hbm, apart_hbm,
             xbuf, ibuf, diffb, predb, stage, sem):
        wid = lax.axis_index("s") * NC + lax.axis_index("c")
        rbase = wid * rows_per_w
        lanes = lax.iota(jnp.int32, L)

        pltpu.sync_copy(i_hbm.at[pl.ds(rbase * 2, rows_per_w * 2)], ibuf)

        pltpu.async_copy(x_hbm.at[pl.ds(rbase * RW, gw)],
                         xbuf.at[pl.ds(0, gw)], sem)

        zf = jnp.zeros((L,), jnp.float32)

        def group_body(g, carry):
            loss_vec, acc_vec = carry
            slot = lax.rem(g, 2)
            sbase = slot * gw
            pltpu.make_async_copy(x_hbm.at[pl.ds(0, gw)],
                                  xbuf.at[pl.ds(sbase, gw)], sem).wait()

            @pl.when(g + 1 < ngroups)
            def _():
                off = rbase * RW + (g + 1) * gw
                pltpu.async_copy(x_hbm.at[pl.ds(off, gw)],
                                 xbuf.at[pl.ds((1 - slot) * gw, gw)], sem)

            av = lanes * RW + sbase

            def dim_body(d, accs):
                base = av + d
                t = plsc.load_gather(xbuf, [base + NH * D])
                out = []
                for i in range(NH):
                    x = plsc.load_gather(xbuf, [base + i * D])
                    e = x - t
                    out.append(accs[i] + e * e)
                return tuple(out)

            accs = lax.fori_loop(0, D, dim_body, (zf,) * NH, unroll=2)

            yt = plsc.load_gather(ibuf, [g * (2 * L) + lanes * 2]) - 8

            best = accs[0]
            besti = jnp.zeros((L,), jnp.int32)
            rowsum = accs[0]
            sel = jnp.where(yt == 0, accs[0], 0.0)
            for i in range(1, NH):
                a = accs[i]
                lt = a < best
                besti = jnp.where(lt, jnp.int32(i), besti)
                best = jnp.where(lt, a, best)
                rowsum = rowsum + a
                sel = sel + jnp.where(yt == i, a, 0.0)

            loss_vec = loss_vec + (2.0 * sel - rowsum)
            acc_vec = acc_vec + jnp.where(besti == yt, 1.0, 0.0)

            for i in range(NH):
                plsc.store_scatter(diffb, [g * (L * NH) + lanes * NH + i],
                                   accs[i])
            plsc.store_scatter(predb, [g * L + lanes], besti)
            return (loss_vec, acc_vec)

        loss_vec, acc_vec = lax.fori_loop(0, ngroups, group_body, (zf, zf))

        pltpu.sync_copy(diffb, diff_hbm.at[pl.ds(rbase * NH, rows_per_w * NH)])
        pltpu.sync_copy(predb, pred_hbm.at[pl.ds(rbase, rows_per_w)])
        stage[...] = loss_vec
        pltpu.sync_copy(stage, lpart_hbm.at[pl.ds(wid * L, L)])
        stage[...] = acc_vec
        pltpu.sync_copy(stage, apart_hbm.at[pl.ds(wid * L, L)])

    return body(x_flat, i_flat)


def _phase2(lpart, apart, B):
    mesh = plsc.VectorSubcoreMesh(core_axis_name="c", subcore_axis_name="s")

    @functools.partial(
        pl.kernel,
        out_type=(
            jax.ShapeDtypeStruct((L,), jnp.float32),
            jax.ShapeDtypeStruct((L,), jnp.float32),
        ),
        mesh=mesh,
        scratch_types=[
            pltpu.VMEM((NW * L,), jnp.float32),
            pltpu.VMEM((NW * L,), jnp.float32),
            pltpu.VMEM((L,), jnp.float32),
            pltpu.VMEM((L,), jnp.float32),
        ],
        compiler_params=pltpu.CompilerParams(needs_layout_passes=False),
    )
    def body(lpart_hbm, apart_hbm, loss_out, acc_out, lbuf, abuf, lst, ast):
        wid = lax.axis_index("s") * NC + lax.axis_index("c")

        @pl.when(wid == 0)
        def _():
            pltpu.sync_copy(lpart_hbm, lbuf)
            pltpu.sync_copy(apart_hbm, abuf)
            ls = jnp.zeros((L,), jnp.float32)
            ac = jnp.zeros((L,), jnp.float32)
            for r in range(NW):
                ls = ls + lbuf[pl.ds(r * L, L)]
                ac = ac + abuf[pl.ds(r * L, L)]
            loss = jnp.sum(ls) * (1.0 / (B * NH))
            acc = jnp.sum(ac) * (1.0 / B)
            lst[...] = jnp.full((L,), loss, jnp.float32)
            ast[...] = jnp.full((L,), acc, jnp.float32)
            pltpu.sync_copy(lst, loss_out)
            pltpu.sync_copy(ast, acc_out)

    return body(lpart, apart)


def kernel(outputs, index):
    B = outputs.shape[0]
    assert outputs.shape[1:] == (H, D)
    assert B % (NW * L) == 0
    x_flat = outputs.reshape(-1)
    i_flat = index.astype(jnp.int32).reshape(-1)
    diff_flat, pred, lpart, apart = _phase1(x_flat, i_flat, B)
    loss16, acc16 = _phase2(lpart, apart, B)
    return diff_flat.reshape(B, NH), pred, loss16[0], acc16[0]
